# Initial kernel scaffold; baseline (speedup 1.0000x reference)
import jax
import jax.numpy as jnp
from jax.experimental import pallas as pl


def kernel(x, edge_index, W1, att_src1, att_dst1, b1, W2, att_src2, att_dst2, b2):
    raise NotImplementedError("write your pallas kernel here")


# speedup vs baseline: 41.3012x; 41.3012x over previous
import functools

import jax
import jax.numpy as jnp
from jax import lax
from jax.experimental import pallas as pl
from jax.experimental.pallas import tpu as pltpu, tpu_sc as plsc

N = 10000
E = 320000
D = 128
H = 8
C = 16
HP = 4
TW = 80
AW = 16
T2W = 16

NC = 2
NS = 16
NW = NC * NS
E_PER = E // NW
K1 = 80
NCH1 = E_PER // K1
K2 = 80
NCH2 = E_PER // K2
NP = 10240
ROWS_PER_TILE = NP // NS
ZR = 128

BR = 2000
GRID = N // BR

_mesh = plsc.VectorSubcoreMesh(core_axis_name="c", subcore_axis_name="s")
_sc_params = pltpu.CompilerParams(use_tc_tiling_on_sc=False)


def _tc1_body(x_ref, w1_ref, m1a_ref, m1b_ref, m2_ref,
              t1a_ref, t1b_ref, a1_ref):
    h = jnp.dot(x_ref[...], w1_ref[...], preferred_element_type=jnp.float32)
    t1a_ref[...] = jnp.dot(h, m1a_ref[...], preferred_element_type=jnp.float32)
    t1b_ref[...] = jnp.dot(h, m1b_ref[...], preferred_element_type=jnp.float32)
    a1_ref[...] = jnp.dot(h, m2_ref[...], preferred_element_type=jnp.float32)


def _tc1(x, W1, M1a, M1b, M2):
    return pl.pallas_call(
        _tc1_body,
        grid=(GRID,),
        in_specs=[
            pl.BlockSpec((BR, D), lambda i: (i, 0)),
            pl.BlockSpec((D, D), lambda i: (0, 0)),
            pl.BlockSpec((D, TW), lambda i: (0, 0)),
            pl.BlockSpec((D, TW), lambda i: (0, 0)),
            pl.BlockSpec((D, AW), lambda i: (0, 0)),
        ],
        out_specs=[
            pl.BlockSpec((BR, TW), lambda i: (i, 0)),
            pl.BlockSpec((BR, TW), lambda i: (i, 0)),
            pl.BlockSpec((BR, AW), lambda i: (i, 0)),
        ],
        out_shape=[
            jax.ShapeDtypeStruct((N, TW), jnp.float32),
            jax.ShapeDtypeStruct((N, TW), jnp.float32),
            jax.ShapeDtypeStruct((N, AW), jnp.float32),
        ],
    )(x, W1, M1a, M1b, M2)


def _sc1_body(t1a_hbm, t1b_hbm, a1_hbm, src_hbm, dst_hbm, out_hbm,
              src_v, dst_v, g1, g2, outb, zbuf, s_sh, sem1, sem2):
    c = lax.axis_index("c")
    s = lax.axis_index("s")
    wid = c * NS + s

    def zrow(r, _):
        for j in range(TW // 16):
            zbuf[r, pl.ds(16 * j, 16)] = jnp.zeros((16,), jnp.float32)
        return 0
    lax.fori_loop(0, ZR, zrow, 0)

    for p, t_hbm in ((0, t1a_hbm), (1, t1b_hbm)):
        for q in range(ROWS_PER_TILE // ZR):
            off = pl.multiple_of(s * ROWS_PER_TILE + q * ZR, 8)
            pltpu.sync_copy(zbuf, s_sh.at[pl.ds(off, ZR)])
        plsc.subcore_barrier()

        def chunk(i, _):
            base = pl.multiple_of(wid * E_PER + i * K1, 8)
            pltpu.sync_copy(src_hbm.at[pl.ds(base, K1)], src_v)
            pltpu.sync_copy(dst_hbm.at[pl.ds(base, K1)], dst_v)
            cp1 = pltpu.async_copy(t_hbm.at[src_v], g1, sem1)
            cp2 = pltpu.async_copy(a1_hbm.at[dst_v], g2, sem2)
            cp1.wait()
            cp2.wait()

            def edge(e, _):
                a_s = g1[e, pl.ds(HP * C, 16)]
                a_d = g2[e, :]
                ev = a_s + a_d
                ev = jnp.maximum(ev, 0.2 * ev)
                gv = jnp.exp(ev)
                outb[e, pl.ds(HP * C, 16)] = gv
                for j in range(HP):
                    gj = gv[HP * p + j]
                    outb[e, pl.ds(16 * j, 16)] = (
                        g1[e, pl.ds(16 * j, 16)] * gj)
                return 0
            lax.fori_loop(0, K1, edge, 0)
            pltpu.sync_copy(outb, s_sh.at[dst_v], add=True)
            return 0
        lax.fori_loop(0, NCH1, chunk, 0)
        plsc.subcore_barrier()

        rb = pl.multiple_of(s * ROWS_PER_TILE, 8)
        pltpu.sync_copy(s_sh.at[pl.ds(rb, ROWS_PER_TILE)],
                        out_hbm.at[p, c, pl.ds(rb, ROWS_PER_TILE)])


_sc1 = functools.partial(
    pl.kernel,
    out_type=jax.ShapeDtypeStruct((2, NC, NP, TW), jnp.float32),
    mesh=_mesh,
    scratch_types=[
        pltpu.VMEM((K1,), jnp.int32),
        pltpu.VMEM((K1,), jnp.int32),
        pltpu.VMEM((K1, TW), jnp.float32),
        pltpu.VMEM((K1, AW), jnp.float32),
        pltpu.VMEM((K1, TW), jnp.float32),
        pltpu.VMEM((ZR, TW), jnp.float32),
        pltpu.VMEM_SHARED((NP, TW), jnp.float32),
        pltpu.SemaphoreType.DMA,
        pltpu.SemaphoreType.DMA,
    ],
    compiler_params=_sc_params,
)(_sc1_body)


def _tc2_body(p_ref, ma_ref, mb_ref, da_ref, db_ref, w2s_ref, w2d_ref,
              b1_ref, t2s_ref, t2d_ref):
    Sa = p_ref[0, 0] + p_ref[0, 1]
    Sb = p_ref[1, 0] + p_ref[1, 1]
    msg = (jnp.dot(Sa, ma_ref[...], preferred_element_type=jnp.float32)
           + jnp.dot(Sb, mb_ref[...], preferred_element_type=jnp.float32))
    den = (jnp.dot(Sa, da_ref[...], preferred_element_type=jnp.float32)
           + jnp.dot(Sb, db_ref[...], preferred_element_type=jnp.float32))
    out1 = jnp.maximum(msg / (den + 1e-16) + b1_ref[...], 0.0)
    one2 = (lax.broadcasted_iota(jnp.int32, (1, T2W), 1) == 2
            ).astype(jnp.float32)
    t2s_ref[...] = (jnp.dot(out1, w2s_ref[...],
                            preferred_element_type=jnp.float32) + one2)
    t2d_ref[...] = jnp.dot(out1, w2d_ref[...],
                           preferred_element_type=jnp.float32)


def _tc2(P1, MA, MB, DA, DB, W2s, W2d, b1r):
    return pl.pallas_call(
        _tc2_body,
        grid=(GRID,),
        in_specs=[
            pl.BlockSpec((2, NC, BR, TW), lambda i: (0, 0, i, 0)),
            pl.BlockSpec((TW, D), lambda i: (0, 0)),
            pl.BlockSpec((TW, D), lambda i: (0, 0)),
            pl.BlockSpec((TW, D), lambda i: (0, 0)),
            pl.BlockSpec((TW, D), lambda i: (0, 0)),
            pl.BlockSpec((D, T2W), lambda i: (0, 0)),
            pl.BlockSpec((D, T2W), lambda i: (0, 0)),
            pl.BlockSpec((1, D), lambda i: (0, 0)),
        ],
        out_specs=[
            pl.BlockSpec((BR, T2W), lambda i: (i, 0)),
            pl.BlockSpec((BR, T2W), lambda i: (i, 0)),
        ],
        out_shape=[
            jax.ShapeDtypeStruct((N, T2W), jnp.float32),
            jax.ShapeDtypeStruct((N, T2W), jnp.float32),
        ],
    )(P1, MA, MB, DA, DB, W2s, W2d, b1r)


def _sc2_body(t2s_hbm, t2d_hbm, src_hbm, dst_hbm, out_hbm,
              src_v, dst_v, g1, g2, outb, zbuf, s_sh, sem1, sem2):
    c = lax.axis_index("c")
    s = lax.axis_index("s")
    wid = c * NS + s

    def zrow(r, _):
        zbuf[r, :] = jnp.zeros((16,), jnp.float32)
        return 0
    lax.fori_loop(0, ZR, zrow, 0)
    for q in range(ROWS_PER_TILE // ZR):
        off = pl.multiple_of(s * ROWS_PER_TILE + q * ZR, 8)
        pltpu.sync_copy(zbuf, s_sh.at[pl.ds(off, ZR)])
    plsc.subcore_barrier()

    def chunk(i, _):
        base = pl.multiple_of(wid * E_PER + i * K2, 8)
        pltpu.sync_copy(src_hbm.at[pl.ds(base, K2)], src_v)
        pltpu.sync_copy(dst_hbm.at[pl.ds(base, K2)], dst_v)
        cp1 = pltpu.async_copy(t2s_hbm.at[src_v], g1, sem1)
        cp2 = pltpu.async_copy(t2d_hbm.at[dst_v], g2, sem2)
        cp1.wait()
        cp2.wait()

        def edge(e, _):
            row_s = g1[e, :]
            row_d = g2[e, :]
            ev = row_s + row_d
            ev = jnp.maximum(ev, 0.2 * ev)
            gv = jnp.exp(ev)
            outb[e, :] = row_s * gv[3]
            return 0
        lax.fori_loop(0, K2, edge, 0)
        pltpu.sync_copy(outb, s_sh.at[dst_v], add=True)
        return 0
    lax.fori_loop(0, NCH2, chunk, 0)
    plsc.subcore_barrier()

    rb = pl.multiple_of(s * ROWS_PER_TILE, 8)
    pltpu.sync_copy(s_sh.at[pl.ds(rb, ROWS_PER_TILE)],
                    out_hbm.at[c, pl.ds(rb, ROWS_PER_TILE)])


_sc2 = functools.partial(
    pl.kernel,
    out_type=jax.ShapeDtypeStruct((NC, NP, T2W), jnp.float32),
    mesh=_mesh,
    scratch_types=[
        pltpu.VMEM((K2,), jnp.int32),
        pltpu.VMEM((K2,), jnp.int32),
        pltpu.VMEM((K2, T2W), jnp.float32),
        pltpu.VMEM((K2, T2W), jnp.float32),
        pltpu.VMEM((K2, T2W), jnp.float32),
        pltpu.VMEM((ZR, T2W), jnp.float32),
        pltpu.VMEM_SHARED((NP, T2W), jnp.float32),
        pltpu.SemaphoreType.DMA,
        pltpu.SemaphoreType.DMA,
    ],
    compiler_params=_sc_params,
)(_sc2_body)


def _tc3_body(p_ref, e01_ref, e22_ref, b2_ref, out_ref):
    S2 = p_ref[0] + p_ref[1]
    num = jnp.dot(S2, e01_ref[...], preferred_element_type=jnp.float32)
    den = jnp.dot(S2, e22_ref[...], preferred_element_type=jnp.float32)
    out_ref[...] = num / (den + 1e-16) + b2_ref[...]


def _tc3(P2, E01, E22, b2r):
    return pl.pallas_call(
        _tc3_body,
        grid=(GRID,),
        in_specs=[
            pl.BlockSpec((NC, BR, T2W), lambda i: (0, i, 0)),
            pl.BlockSpec((T2W, 2), lambda i: (0, 0)),
            pl.BlockSpec((T2W, 2), lambda i: (0, 0)),
            pl.BlockSpec((1, 2), lambda i: (0, 0)),
        ],
        out_specs=pl.BlockSpec((BR, 2), lambda i: (i, 0)),
        out_shape=jax.ShapeDtypeStruct((N, 2), jnp.float32),
    )(P2, E01, E22, b2r)


def kernel(x, edge_index, W1, att_src1, att_dst1, b1, W2, att_src2,
           att_dst2, b2):
    src = edge_index[0].astype(jnp.int32)
    dst = edge_index[1].astype(jnp.int32)

    eye8 = jnp.eye(H, dtype=jnp.float32)
    eye64 = jnp.eye(64, dtype=jnp.float32)
    z64 = jnp.zeros((64, 64), jnp.float32)
    As = (att_src1[:, :, None] * eye8[:, None, :]).reshape(D, H)
    Ad = (att_dst1[:, :, None] * eye8[:, None, :]).reshape(D, H)
    M1a = jnp.concatenate(
        [jnp.concatenate([eye64, z64], axis=0),
         jnp.concatenate([As[:, 0:HP], jnp.zeros((D, 12), jnp.float32)],
                         axis=1)], axis=1)
    M1b = jnp.concatenate(
        [jnp.concatenate([z64, eye64], axis=0),
         jnp.concatenate([jnp.zeros((D, HP), jnp.float32), As[:, HP:H],
                          jnp.zeros((D, 8), jnp.float32)], axis=1)], axis=1)
    M2 = jnp.concatenate([Ad, jnp.zeros((D, AW - H), jnp.float32)], axis=1)
    MA = jnp.concatenate(
        [jnp.concatenate([eye64, z64], axis=1),
         jnp.zeros((16, D), jnp.float32)], axis=0)
    MB = jnp.concatenate(
        [jnp.concatenate([z64, eye64], axis=1),
         jnp.zeros((16, D), jnp.float32)], axis=0)
    rep8 = jnp.repeat(eye8, C, axis=1)
    DA = jnp.concatenate(
        [jnp.zeros((64, D), jnp.float32), rep8[0:HP],
         jnp.zeros((12, D), jnp.float32)], axis=0)
    DB = jnp.concatenate(
        [jnp.zeros((68, D), jnp.float32), rep8[HP:H],
         jnp.zeros((8, D), jnp.float32)], axis=0)
    W2s = jnp.concatenate(
        [W2, jnp.zeros((D, 1), jnp.float32), (W2 @ att_src2[0])[:, None],
         jnp.zeros((D, T2W - 4), jnp.float32)], axis=1)
    W2d = jnp.concatenate(
        [jnp.zeros((D, 3), jnp.float32), (W2 @ att_dst2[0])[:, None],
         jnp.zeros((D, T2W - 4), jnp.float32)], axis=1)
    E01 = jnp.zeros((T2W, 2), jnp.float32).at[0, 0].set(1.0).at[1, 1].set(1.0)
    E22 = jnp.zeros((T2W, 2), jnp.float32).at[2, 0].set(1.0).at[2, 1].set(1.0)

    T1a, T1b, A1 = _tc1(x, W1, M1a, M1b, M2)
    P1 = _sc1(T1a, T1b, A1, src, dst)
    T2s, T2d = _tc2(P1, MA, MB, DA, DB, W2s, W2d, b1[None, :])
    P2 = _sc2(T2s, T2d, src, dst)
    return _tc3(P2, E01, E22, b2[None, :])
